# baseline (speedup 1.0000x reference)
import functools

import jax
import jax.numpy as jnp
from jax import lax
from jax.experimental import pallas as pl
from jax.experimental.pallas import tpu as pltpu
from jax.experimental.pallas import tpu_sc as plsc

NC = 2
NS = 16
NW = NC * NS
BLK = 256


def _router_body(x_ref, wg_ref, bg_ref, wn_ref, bn_ref, noise_ref,
                 e0_ref, e1_ref, g0_ref, g1_ref):
    x = x_ref[...]
    logits = jnp.dot(x, wg_ref[...], preferred_element_type=jnp.float32) + bg_ref[...]
    nlog = jnp.dot(x, wn_ref[...], preferred_element_type=jnp.float32) + bn_ref[...]
    sp = jnp.maximum(nlog, 0.0) + jnp.log1p(jnp.exp(-jnp.abs(nlog)))
    noisy = logits + noise_ref[...] * sp

    E = noisy.shape[-1]
    col = lax.broadcasted_iota(jnp.int32, noisy.shape, 1)
    m1 = jnp.max(noisy, axis=1, keepdims=True)
    i1 = jnp.min(jnp.where(noisy == m1, col, E), axis=1, keepdims=True)
    masked = jnp.where(col == i1, -jnp.inf, noisy)
    m2 = jnp.max(masked, axis=1, keepdims=True)
    i2 = jnp.min(jnp.where(masked == m2, col, E), axis=1, keepdims=True)
    w2 = jnp.exp(m2 - m1)
    denom = 1.0 + w2
    e0_ref[...] = i1
    e1_ref[...] = i2
    g0_ref[...] = 1.0 / denom
    g1_ref[...] = w2 / denom


def _ranking_body(es_ref, p_ref, be_ref, nb_ref, r_ref, carry_ref, off_ref,
                  *, n_exp, blk, nblocks, sblk):
    ph = pl.program_id(0)
    j = pl.program_id(1)

    es = es_ref[...]
    lane = lax.broadcasted_iota(jnp.int32, (sblk, n_exp), 1)
    oh = (es == lane).astype(jnp.float32)

    @pl.when(ph == 0)
    def _():
        r = lax.broadcasted_iota(jnp.int32, (sblk, sblk), 0)
        c = lax.broadcasted_iota(jnp.int32, (sblk, sblk), 1)
        tril = (r >= c).astype(jnp.float32)
        csum = jnp.dot(tril, oh, preferred_element_type=jnp.float32)
        carry = jnp.where(j == 0, 0.0, carry_ref[...])
        rank = jnp.sum(oh * (csum + carry), axis=1, keepdims=True) - 1.0
        r_ref[pl.ds(j * sblk, sblk), :] = rank
        carry_ref[...] = carry + csum[sblk - 1:sblk, :]

    @pl.when((ph == 1) & (j == 0))
    def _():
        counts = carry_ref[...]
        nblk = jnp.floor((counts + (blk - 1)) / blk)
        pc = nblk * blk
        a = lax.broadcasted_iota(jnp.int32, (n_exp, n_exp), 0)
        b = lax.broadcasted_iota(jnp.int32, (n_exp, n_exp), 1)
        strict = (a < b).astype(jnp.float32)
        off = jnp.dot(pc, strict, preferred_element_type=jnp.float32)
        off_ref[...] = off
        ob = off / blk
        jj = lax.broadcasted_iota(jnp.int32, (nblocks, n_exp), 0).astype(jnp.float32)
        be = jnp.sum((jj >= ob).astype(jnp.float32), axis=1, keepdims=True) - 1.0
        be_ref[...] = be.astype(jnp.int32)
        nb_ref[...] = jnp.sum(nblk, axis=1, keepdims=True).astype(jnp.int32)

    @pl.when(ph == 1)
    def _():
        off = off_ref[...]
        pos = jnp.sum(oh * off, axis=1, keepdims=True) + r_ref[pl.ds(j * sblk, sblk), :]
        p_ref[...] = pos.astype(jnp.int32)


def _dispatch_body(pos_hbm, g_hbm, x_hbm, xs_hbm, gs_hbm,
                   zi_v, zf_v, pos_v, tok_v, gch_v, gidx_v, gsl_v, rows_v,
                   sp_tok, sp_g, sem, *, n_tok, spad, d_model):
    cid = lax.axis_index("c")
    sid = lax.axis_index("s")
    wid = sid * NC + cid

    slots_per_tile = 2 * n_tok // NS
    zrange = spad // NS
    rows_per_tile = spad // NW
    gchunk = rows_v.shape[0]

    for i in range(zrange // 16):
        zi_v[pl.ds(i * 16, 16)] = jnp.zeros((16,), jnp.int32)
        zf_v[pl.ds(i * 16, 16)] = jnp.zeros((16,), jnp.float32)
    pltpu.sync_copy(zi_v, sp_tok.at[pl.ds(sid * zrange, zrange)])
    pltpu.sync_copy(zf_v, sp_g.at[pl.ds(sid * zrange, zrange)])

    base = sid * slots_per_tile
    pltpu.sync_copy(pos_hbm.at[pl.ds(base, slots_per_tile)], pos_v)
    pltpu.sync_copy(g_hbm.at[pl.ds(base, slots_per_tile)], gch_v)
    for i in range(slots_per_tile // 16):
        s = base + i * 16 + lax.iota(jnp.int32, 16)
        tok_v[pl.ds(i * 16, 16)] = jnp.where(s < n_tok, s, s - n_tok)

    plsc.subcore_barrier()
    pltpu.sync_copy(tok_v, sp_tok.at[pos_v], add=True)
    pltpu.sync_copy(gch_v, sp_g.at[pos_v], add=True)
    plsc.subcore_barrier()

    rbase = wid * rows_per_tile
    pltpu.sync_copy(sp_tok.at[pl.ds(rbase, rows_per_tile)], gidx_v)
    pltpu.sync_copy(sp_g.at[pl.ds(rbase, rows_per_tile)], gsl_v)
    pltpu.sync_copy(gsl_v, gs_hbm.at[pl.ds(rbase, rows_per_tile)])
    pltpu.async_copy(x_hbm.at[gidx_v], rows_v, sem).wait()
    pltpu.sync_copy(rows_v, xs_hbm.at[pl.ds(rbase, rows_per_tile)])


def _ffn_body(be_ref, nb_ref, xs_ref, gs_ref, w1_ref, b1_ref, w2_ref, b2_ref,
              yo_ref, w1b_ref, w2b_ref):
    i = pl.program_id(0)
    active = i < nb_ref[0]
    ecur = be_ref[i]
    eprev = be_ref[jnp.maximum(i - 1, 0)]
    need_cast = jnp.logical_or(i == 0, ecur != eprev)

    @pl.when(active & need_cast)
    def _():
        w1b_ref[...] = w1_ref[0].astype(jnp.bfloat16)
        w2b_ref[...] = w2_ref[0].astype(jnp.bfloat16)

    @pl.when(active)
    def _():
        xb = xs_ref[...]
        h = jnp.dot(xb, w1b_ref[...], preferred_element_type=jnp.float32) + b1_ref[0, 0]
        hb = jnp.maximum(h, 0.0).astype(jnp.bfloat16)
        eo = jnp.dot(hb, w2b_ref[...], preferred_element_type=jnp.float32) + b2_ref[0, 0]
        yo_ref[...] = eo * gs_ref[...]


def _combine_body(yo_hbm, p0_hbm, p1_hbm, out_hbm, p0_v, p1_v, r0_v, r1_v,
                  sem, *, n_tok):
    cid = lax.axis_index("c")
    sid = lax.axis_index("s")
    wid = sid * NC + cid
    tpt = n_tok // NW
    base = wid * tpt
    half = r0_v.shape[0]
    d_model = r0_v.shape[1]
    pltpu.sync_copy(p0_hbm.at[pl.ds(base, tpt)], p0_v)
    pltpu.sync_copy(p1_hbm.at[pl.ds(base, tpt)], p1_v)
    for c in range(tpt // half):
        d0 = pltpu.async_copy(yo_hbm.at[p0_v.at[pl.ds(c * half, half)]], r0_v, sem)
        d1 = pltpu.async_copy(yo_hbm.at[p1_v.at[pl.ds(c * half, half)]], r1_v, sem)
        d0.wait()
        d1.wait()

        def row(r, _):
            def chunk(k, _):
                sl = pl.ds(k * 16, 16)
                r0_v[r, sl] = r0_v[r, sl] + r1_v[r, sl]
                return 0
            return lax.fori_loop(0, d_model // 16, chunk, 0)

        lax.fori_loop(0, half, row, 0)
        pltpu.sync_copy(r0_v, out_hbm.at[pl.ds(base + c * half, half)])


def kernel(x, Wg, bg, Wn, bn, W1, b1, W2, b2):
    N, D = x.shape
    E = Wg.shape[1]
    H = W1.shape[2]
    S = 2 * N
    NB = S // BLK + E
    SPAD = NB * BLK

    noise = jax.random.normal(jax.random.key(42), (N, E), jnp.float32)

    tb = 256
    e0, e1, g0, g1 = pl.pallas_call(
        _router_body,
        grid=(N // tb,),
        in_specs=[
            pl.BlockSpec((tb, D), lambda t: (t, 0)),
            pl.BlockSpec((D, E), lambda t: (0, 0)),
            pl.BlockSpec((1, E), lambda t: (0, 0)),
            pl.BlockSpec((D, E), lambda t: (0, 0)),
            pl.BlockSpec((1, E), lambda t: (0, 0)),
            pl.BlockSpec((tb, E), lambda t: (t, 0)),
        ],
        out_specs=[
            pl.BlockSpec((tb, 1), lambda t: (t, 0)),
            pl.BlockSpec((tb, 1), lambda t: (t, 0)),
            pl.BlockSpec((tb, 1), lambda t: (t, 0)),
            pl.BlockSpec((tb, 1), lambda t: (t, 0)),
        ],
        out_shape=[
            jax.ShapeDtypeStruct((N, 1), jnp.int32),
            jax.ShapeDtypeStruct((N, 1), jnp.int32),
            jax.ShapeDtypeStruct((N, 1), jnp.float32),
            jax.ShapeDtypeStruct((N, 1), jnp.float32),
        ],
    )(x, Wg, bg.reshape(1, E), Wn, bn.reshape(1, E), noise)

    es = jnp.concatenate([e0, e1], axis=0)
    gall = jnp.concatenate([g0, g1], axis=0)

    sblk = 256
    pos, be, nbact = pl.pallas_call(
        functools.partial(_ranking_body, n_exp=E, blk=BLK, nblocks=NB, sblk=sblk),
        grid=(2, S // sblk),
        in_specs=[pl.BlockSpec((sblk, 1), lambda ph, j: (j, 0))],
        out_specs=[
            pl.BlockSpec((sblk, 1), lambda ph, j: (j, 0)),
            pl.BlockSpec((NB, 1), lambda ph, j: (0, 0)),
            pl.BlockSpec((1, 1), lambda ph, j: (0, 0)),
        ],
        out_shape=[
            jax.ShapeDtypeStruct((S, 1), jnp.int32),
            jax.ShapeDtypeStruct((NB, 1), jnp.int32),
            jax.ShapeDtypeStruct((1, 1), jnp.int32),
        ],
        scratch_shapes=[
            pltpu.VMEM((S, 1), jnp.float32),
            pltpu.VMEM((1, E), jnp.float32),
            pltpu.VMEM((1, E), jnp.float32),
        ],
    )(es)

    pos_flat = pos.reshape(S)
    g_flat = gall.reshape(S)

    mesh = plsc.VectorSubcoreMesh(core_axis_name="c", subcore_axis_name="s",
                                  num_cores=NC, num_subcores=NS)
    xb = x.astype(jnp.bfloat16)
    x32 = jax.lax.bitcast_convert_type(xb.reshape(N, D // 2, 2), jnp.int32)
    xs32, gsort = pl.kernel(
        functools.partial(_dispatch_body, n_tok=N, spad=SPAD, d_model=D),
        out_type=[
            jax.ShapeDtypeStruct((SPAD, D // 2), jnp.int32),
            jax.ShapeDtypeStruct((SPAD,), jnp.float32),
        ],
        mesh=mesh,
        scratch_types=[
            pltpu.VMEM((SPAD // NS,), jnp.int32),
            pltpu.VMEM((SPAD // NS,), jnp.float32),
            pltpu.VMEM((S // NS,), jnp.int32),
            pltpu.VMEM((S // NS,), jnp.int32),
            pltpu.VMEM((S // NS,), jnp.float32),
            pltpu.VMEM((SPAD // NW,), jnp.int32),
            pltpu.VMEM((SPAD // NW,), jnp.float32),
            pltpu.VMEM((SPAD // NW, D // 2), jnp.int32),
            pltpu.VMEM_SHARED((SPAD,), jnp.int32),
            pltpu.VMEM_SHARED((SPAD,), jnp.float32),
            pltpu.SemaphoreType.DMA,
        ],
    )(pos_flat, g_flat, x32)
    xs = jax.lax.bitcast_convert_type(xs32, jnp.bfloat16).reshape(SPAD, D)

    grid_spec = pltpu.PrefetchScalarGridSpec(
        num_scalar_prefetch=2,
        grid=(NB,),
        in_specs=[
            pl.BlockSpec((BLK, D), lambda i, be_r, nb_r: (i, 0)),
            pl.BlockSpec((BLK, 1), lambda i, be_r, nb_r: (i, 0)),
            pl.BlockSpec((1, D, H), lambda i, be_r, nb_r: (be_r[i], 0, 0)),
            pl.BlockSpec((1, 1, H), lambda i, be_r, nb_r: (be_r[i], 0, 0)),
            pl.BlockSpec((1, H, D), lambda i, be_r, nb_r: (be_r[i], 0, 0)),
            pl.BlockSpec((1, 1, D), lambda i, be_r, nb_r: (be_r[i], 0, 0)),
        ],
        out_specs=pl.BlockSpec((BLK, D), lambda i, be_r, nb_r: (i, 0)),
        scratch_shapes=[
            pltpu.VMEM((D, H), jnp.bfloat16),
            pltpu.VMEM((H, D), jnp.bfloat16),
        ],
    )
    yo = pl.pallas_call(
        _ffn_body,
        grid_spec=grid_spec,
        out_shape=jax.ShapeDtypeStruct((SPAD, D), jnp.float32),
    )(be.reshape(NB), nbact.reshape(1), xs, gsort.reshape(SPAD, 1),
      W1, b1.reshape(E, 1, H), W2, b2.reshape(E, 1, D))

    out = pl.kernel(
        functools.partial(_combine_body, n_tok=N),
        out_type=jax.ShapeDtypeStruct((N, D), jnp.float32),
        mesh=mesh,
        scratch_types=[
            pltpu.VMEM((N // NW,), jnp.int32),
            pltpu.VMEM((N // NW,), jnp.int32),
            pltpu.VMEM((N // NW // 2, D), jnp.float32),
            pltpu.VMEM((N // NW // 2, D), jnp.float32),
            pltpu.SemaphoreType.DMA,
        ],
    )(yo, pos_flat[:N], pos_flat[N:])
    return out


# speedup vs baseline: 1.4837x; 1.4837x over previous
import functools

import jax
import jax.numpy as jnp
from jax import lax
from jax.experimental import pallas as pl
from jax.experimental.pallas import tpu as pltpu
from jax.experimental.pallas import tpu_sc as plsc

NC = 2
NS = 16
NW = NC * NS
BLK = 256


def _router_body(x_ref, wg_ref, bg_ref, wn_ref, bn_ref, noise_ref,
                 e0_ref, e1_ref, g0_ref, g1_ref):
    x = x_ref[...]
    logits = jnp.dot(x, wg_ref[...], preferred_element_type=jnp.float32) + bg_ref[...]
    nlog = jnp.dot(x, wn_ref[...], preferred_element_type=jnp.float32) + bn_ref[...]
    sp = jnp.maximum(nlog, 0.0) + jnp.log1p(jnp.exp(-jnp.abs(nlog)))
    noisy = logits + noise_ref[...] * sp

    E = noisy.shape[-1]
    col = lax.broadcasted_iota(jnp.int32, noisy.shape, 1)
    m1 = jnp.max(noisy, axis=1, keepdims=True)
    i1 = jnp.min(jnp.where(noisy == m1, col, E), axis=1, keepdims=True)
    masked = jnp.where(col == i1, -jnp.inf, noisy)
    m2 = jnp.max(masked, axis=1, keepdims=True)
    i2 = jnp.min(jnp.where(masked == m2, col, E), axis=1, keepdims=True)
    w2 = jnp.exp(m2 - m1)
    denom = 1.0 + w2
    e0_ref[...] = i1
    e1_ref[...] = i2
    g0_ref[...] = 1.0 / denom
    g1_ref[...] = w2 / denom


def _ranking_body(es_ref, p_ref, be_ref, nb_ref, r_ref, carry_ref, off_ref,
                  *, n_exp, blk, nblocks, sblk):
    ph = pl.program_id(0)
    j = pl.program_id(1)

    es = es_ref[...]
    lane = lax.broadcasted_iota(jnp.int32, (sblk, n_exp), 1)
    oh = (es == lane).astype(jnp.float32)

    @pl.when(ph == 0)
    def _():
        r = lax.broadcasted_iota(jnp.int32, (sblk, sblk), 0)
        c = lax.broadcasted_iota(jnp.int32, (sblk, sblk), 1)
        tril = (r >= c).astype(jnp.float32)
        csum = jnp.dot(tril, oh, preferred_element_type=jnp.float32)
        carry = jnp.where(j == 0, 0.0, carry_ref[...])
        rank = jnp.sum(oh * (csum + carry), axis=1, keepdims=True) - 1.0
        r_ref[pl.ds(j * sblk, sblk), :] = rank
        carry_ref[...] = carry + csum[sblk - 1:sblk, :]

    @pl.when((ph == 1) & (j == 0))
    def _():
        counts = carry_ref[...]
        nblk = jnp.floor((counts + (blk - 1)) / blk)
        pc = nblk * blk
        a = lax.broadcasted_iota(jnp.int32, (n_exp, n_exp), 0)
        b = lax.broadcasted_iota(jnp.int32, (n_exp, n_exp), 1)
        strict = (a < b).astype(jnp.float32)
        off = jnp.dot(pc, strict, preferred_element_type=jnp.float32)
        off_ref[...] = off
        ob = off / blk
        jj = lax.broadcasted_iota(jnp.int32, (nblocks, n_exp), 0).astype(jnp.float32)
        be = jnp.sum((jj >= ob).astype(jnp.float32), axis=1, keepdims=True) - 1.0
        be_ref[...] = be.astype(jnp.int32)
        nb_ref[...] = jnp.sum(nblk, axis=1, keepdims=True).astype(jnp.int32)

    @pl.when(ph == 1)
    def _():
        off = off_ref[...]
        pos = jnp.sum(oh * off, axis=1, keepdims=True) + r_ref[pl.ds(j * sblk, sblk), :]
        p_ref[...] = pos.astype(jnp.int32)


def _dispatch_body(pos_hbm, g_hbm, x_hbm, xs_hbm, gs_hbm,
                   zi_v, zf_v, pos_v, tok_v, gch_v, gidx_v, gsl_v,
                   rows_a, rows_b, sp_tok, sp_g, sem, *, n_tok, spad, d_model):
    cid = lax.axis_index("c")
    sid = lax.axis_index("s")
    wid = sid * NC + cid

    slots_per_tile = 2 * n_tok // NS
    zrange = spad // NS
    rows_per_tile = spad // NW
    gchunk = rows_a.shape[0]

    for i in range(zrange // 16):
        zi_v[pl.ds(i * 16, 16)] = jnp.zeros((16,), jnp.int32)
        zf_v[pl.ds(i * 16, 16)] = jnp.zeros((16,), jnp.float32)
    pltpu.sync_copy(zi_v, sp_tok.at[pl.ds(sid * zrange, zrange)])
    pltpu.sync_copy(zf_v, sp_g.at[pl.ds(sid * zrange, zrange)])

    base = sid * slots_per_tile
    pltpu.sync_copy(pos_hbm.at[pl.ds(base, slots_per_tile)], pos_v)
    pltpu.sync_copy(g_hbm.at[pl.ds(base, slots_per_tile)], gch_v)
    for i in range(slots_per_tile // 16):
        s = base + i * 16 + lax.iota(jnp.int32, 16)
        tok_v[pl.ds(i * 16, 16)] = jnp.where(s < n_tok, s, s - n_tok)

    plsc.subcore_barrier()
    pltpu.sync_copy(tok_v, sp_tok.at[pos_v], add=True)
    pltpu.sync_copy(gch_v, sp_g.at[pos_v], add=True)
    plsc.subcore_barrier()

    rbase = wid * rows_per_tile
    pltpu.sync_copy(sp_tok.at[pl.ds(rbase, rows_per_tile)], gidx_v)
    pltpu.sync_copy(sp_g.at[pl.ds(rbase, rows_per_tile)], gsl_v)
    pltpu.sync_copy(gsl_v, gs_hbm.at[pl.ds(rbase, rows_per_tile)])
    nch = rows_per_tile // gchunk
    bufs = (rows_a, rows_b)
    d = pltpu.async_copy(x_hbm.at[gidx_v.at[pl.ds(0, gchunk)]], rows_a, sem)
    for c in range(nch):
        d.wait()
        if c + 1 < nch:
            d = pltpu.async_copy(
                x_hbm.at[gidx_v.at[pl.ds((c + 1) * gchunk, gchunk)]],
                bufs[(c + 1) % 2], sem)
        pltpu.sync_copy(bufs[c % 2], xs_hbm.at[pl.ds(rbase + c * gchunk, gchunk)])


def _ffn_body(be_ref, nb_ref, xs_ref, gs_ref, w1_ref, b1_ref, w2_ref, b2_ref,
              yo_ref, w1b_ref, w2b_ref):
    i = pl.program_id(0)
    active = i < nb_ref[0]
    ecur = be_ref[i]
    eprev = be_ref[jnp.maximum(i - 1, 0)]
    need_cast = jnp.logical_or(i == 0, ecur != eprev)

    @pl.when(active & need_cast)
    def _():
        w1b_ref[...] = w1_ref[0].astype(jnp.bfloat16)
        w2b_ref[...] = w2_ref[0].astype(jnp.bfloat16)

    @pl.when(active)
    def _():
        xb = xs_ref[...].astype(jnp.bfloat16)
        h = jnp.dot(xb, w1b_ref[...], preferred_element_type=jnp.float32) + b1_ref[0, 0]
        hb = jnp.maximum(h, 0.0).astype(jnp.bfloat16)
        eo = jnp.dot(hb, w2b_ref[...], preferred_element_type=jnp.float32) + b2_ref[0, 0]
        yo_ref[...] = eo * gs_ref[...]


def _combine_body(yo_hbm, p0_hbm, p1_hbm, out_hbm, p0_v, p1_v, r0_v, r1_v,
                  sem, *, n_tok):
    cid = lax.axis_index("c")
    sid = lax.axis_index("s")
    wid = sid * NC + cid
    tpt = n_tok // NW
    base = wid * tpt
    half = r0_v.shape[0]
    d_model = r0_v.shape[1]
    pltpu.sync_copy(p0_hbm.at[pl.ds(base, tpt)], p0_v)
    pltpu.sync_copy(p1_hbm.at[pl.ds(base, tpt)], p1_v)
    for c in range(tpt // half):
        d0 = pltpu.async_copy(yo_hbm.at[p0_v.at[pl.ds(c * half, half)]], r0_v, sem)
        d1 = pltpu.async_copy(yo_hbm.at[p1_v.at[pl.ds(c * half, half)]], r1_v, sem)
        d0.wait()
        d1.wait()

        def row(r, _):
            def chunk(k, _):
                sl = pl.ds(k * 16, 16)
                r0_v[r, sl] = r0_v[r, sl] + r1_v[r, sl]
                return 0
            return lax.fori_loop(0, d_model // 16, chunk, 0)

        lax.fori_loop(0, half, row, 0)
        pltpu.sync_copy(r0_v, out_hbm.at[pl.ds(base + c * half, half)])


def kernel(x, Wg, bg, Wn, bn, W1, b1, W2, b2):
    N, D = x.shape
    E = Wg.shape[1]
    H = W1.shape[2]
    S = 2 * N
    NB = S // BLK + E
    SPAD = NB * BLK

    noise = jax.random.normal(jax.random.key(42), (N, E), jnp.float32)

    tb = 256
    e0, e1, g0, g1 = pl.pallas_call(
        _router_body,
        grid=(N // tb,),
        in_specs=[
            pl.BlockSpec((tb, D), lambda t: (t, 0)),
            pl.BlockSpec((D, E), lambda t: (0, 0)),
            pl.BlockSpec((1, E), lambda t: (0, 0)),
            pl.BlockSpec((D, E), lambda t: (0, 0)),
            pl.BlockSpec((1, E), lambda t: (0, 0)),
            pl.BlockSpec((tb, E), lambda t: (t, 0)),
        ],
        out_specs=[
            pl.BlockSpec((tb, 1), lambda t: (t, 0)),
            pl.BlockSpec((tb, 1), lambda t: (t, 0)),
            pl.BlockSpec((tb, 1), lambda t: (t, 0)),
            pl.BlockSpec((tb, 1), lambda t: (t, 0)),
        ],
        out_shape=[
            jax.ShapeDtypeStruct((N, 1), jnp.int32),
            jax.ShapeDtypeStruct((N, 1), jnp.int32),
            jax.ShapeDtypeStruct((N, 1), jnp.float32),
            jax.ShapeDtypeStruct((N, 1), jnp.float32),
        ],
    )(x, Wg, bg.reshape(1, E), Wn, bn.reshape(1, E), noise)

    es = jnp.concatenate([e0, e1], axis=0)
    gall = jnp.concatenate([g0, g1], axis=0)

    sblk = 256
    pos, be, nbact = pl.pallas_call(
        functools.partial(_ranking_body, n_exp=E, blk=BLK, nblocks=NB, sblk=sblk),
        grid=(2, S // sblk),
        in_specs=[pl.BlockSpec((sblk, 1), lambda ph, j: (j, 0))],
        out_specs=[
            pl.BlockSpec((sblk, 1), lambda ph, j: (j, 0)),
            pl.BlockSpec((NB, 1), lambda ph, j: (0, 0)),
            pl.BlockSpec((1, 1), lambda ph, j: (0, 0)),
        ],
        out_shape=[
            jax.ShapeDtypeStruct((S, 1), jnp.int32),
            jax.ShapeDtypeStruct((NB, 1), jnp.int32),
            jax.ShapeDtypeStruct((1, 1), jnp.int32),
        ],
        scratch_shapes=[
            pltpu.VMEM((S, 1), jnp.float32),
            pltpu.VMEM((1, E), jnp.float32),
            pltpu.VMEM((1, E), jnp.float32),
        ],
    )(es)

    pos_flat = pos.reshape(S)
    g_flat = gall.reshape(S)

    mesh = plsc.VectorSubcoreMesh(core_axis_name="c", subcore_axis_name="s",
                                  num_cores=NC, num_subcores=NS)
    xs, gsort = pl.kernel(
        functools.partial(_dispatch_body, n_tok=N, spad=SPAD, d_model=D),
        out_type=[
            jax.ShapeDtypeStruct((SPAD, D), jnp.float32),
            jax.ShapeDtypeStruct((SPAD,), jnp.float32),
        ],
        mesh=mesh,
        scratch_types=[
            pltpu.VMEM((SPAD // NS,), jnp.int32),
            pltpu.VMEM((SPAD // NS,), jnp.float32),
            pltpu.VMEM((S // NS,), jnp.int32),
            pltpu.VMEM((S // NS,), jnp.int32),
            pltpu.VMEM((S // NS,), jnp.float32),
            pltpu.VMEM((SPAD // NW,), jnp.int32),
            pltpu.VMEM((SPAD // NW,), jnp.float32),
            pltpu.VMEM((SPAD // NW // 4, D), jnp.float32),
            pltpu.VMEM((SPAD // NW // 4, D), jnp.float32),
            pltpu.VMEM_SHARED((SPAD,), jnp.int32),
            pltpu.VMEM_SHARED((SPAD,), jnp.float32),
            pltpu.SemaphoreType.DMA,
        ],
    )(pos_flat, g_flat, x)

    grid_spec = pltpu.PrefetchScalarGridSpec(
        num_scalar_prefetch=2,
        grid=(NB,),
        in_specs=[
            pl.BlockSpec((BLK, D), lambda i, be_r, nb_r: (i, 0)),
            pl.BlockSpec((BLK, 1), lambda i, be_r, nb_r: (i, 0)),
            pl.BlockSpec((1, D, H), lambda i, be_r, nb_r: (be_r[i], 0, 0)),
            pl.BlockSpec((1, 1, H), lambda i, be_r, nb_r: (be_r[i], 0, 0)),
            pl.BlockSpec((1, H, D), lambda i, be_r, nb_r: (be_r[i], 0, 0)),
            pl.BlockSpec((1, 1, D), lambda i, be_r, nb_r: (be_r[i], 0, 0)),
        ],
        out_specs=pl.BlockSpec((BLK, D), lambda i, be_r, nb_r: (i, 0)),
        scratch_shapes=[
            pltpu.VMEM((D, H), jnp.bfloat16),
            pltpu.VMEM((H, D), jnp.bfloat16),
        ],
    )
    yo = pl.pallas_call(
        _ffn_body,
        grid_spec=grid_spec,
        out_shape=jax.ShapeDtypeStruct((SPAD, D), jnp.float32),
    )(be.reshape(NB), nbact.reshape(1), xs, gsort.reshape(SPAD, 1),
      W1, b1.reshape(E, 1, H), W2, b2.reshape(E, 1, D))

    out = pl.kernel(
        functools.partial(_combine_body, n_tok=N),
        out_type=jax.ShapeDtypeStruct((N, D), jnp.float32),
        mesh=mesh,
        scratch_types=[
            pltpu.VMEM((N // NW,), jnp.int32),
            pltpu.VMEM((N // NW,), jnp.int32),
            pltpu.VMEM((N // NW // 2, D), jnp.float32),
            pltpu.VMEM((N // NW // 2, D), jnp.float32),
            pltpu.SemaphoreType.DMA,
        ],
    )(yo, pos_flat[:N], pos_flat[N:])
    return out


# speedup vs baseline: 1.7513x; 1.1804x over previous
import functools

import jax
import jax.numpy as jnp
from jax import lax
from jax.experimental import pallas as pl
from jax.experimental.pallas import tpu as pltpu
from jax.experimental.pallas import tpu_sc as plsc

NC = 2
NS = 16
NW = NC * NS
BLK = 256


def _router_body(x_ref, wg_ref, bg_ref, wn_ref, bn_ref, noise_ref,
                 e0_ref, e1_ref, g0_ref, g1_ref):
    x = x_ref[...]
    logits = jnp.dot(x, wg_ref[...], preferred_element_type=jnp.float32) + bg_ref[...]
    nlog = jnp.dot(x, wn_ref[...], preferred_element_type=jnp.float32) + bn_ref[...]
    sp = jnp.maximum(nlog, 0.0) + jnp.log1p(jnp.exp(-jnp.abs(nlog)))
    noisy = logits + noise_ref[...] * sp

    E = noisy.shape[-1]
    col = lax.broadcasted_iota(jnp.int32, noisy.shape, 1)
    m1 = jnp.max(noisy, axis=1, keepdims=True)
    i1 = jnp.min(jnp.where(noisy == m1, col, E), axis=1, keepdims=True)
    masked = jnp.where(col == i1, -jnp.inf, noisy)
    m2 = jnp.max(masked, axis=1, keepdims=True)
    i2 = jnp.min(jnp.where(masked == m2, col, E), axis=1, keepdims=True)
    w2 = jnp.exp(m2 - m1)
    denom = 1.0 + w2
    e0_ref[...] = i1
    e1_ref[...] = i2
    g0_ref[...] = 1.0 / denom
    g1_ref[...] = w2 / denom


def _ranking_body(es_ref, p_ref, be_ref, nb_ref, r_ref, carry_ref, off_ref,
                  *, n_exp, blk, nblocks, sblk):
    ph = pl.program_id(0)
    j = pl.program_id(1)

    es = es_ref[...]
    lane = lax.broadcasted_iota(jnp.int32, (sblk, n_exp), 1)
    oh = (es == lane).astype(jnp.float32)

    @pl.when(ph == 0)
    def _():
        r = lax.broadcasted_iota(jnp.int32, (sblk, sblk), 0)
        c = lax.broadcasted_iota(jnp.int32, (sblk, sblk), 1)
        tril = (r >= c).astype(jnp.float32)
        csum = jnp.dot(tril, oh, preferred_element_type=jnp.float32)
        carry = jnp.where(j == 0, 0.0, carry_ref[...])
        rank = jnp.sum(oh * (csum + carry), axis=1, keepdims=True) - 1.0
        r_ref[pl.ds(j * sblk, sblk), :] = rank
        carry_ref[...] = carry + csum[sblk - 1:sblk, :]

    @pl.when((ph == 1) & (j == 0))
    def _():
        counts = carry_ref[...]
        nblk = jnp.floor((counts + (blk - 1)) / blk)
        pc = nblk * blk
        a = lax.broadcasted_iota(jnp.int32, (n_exp, n_exp), 0)
        b = lax.broadcasted_iota(jnp.int32, (n_exp, n_exp), 1)
        strict = (a < b).astype(jnp.float32)
        off = jnp.dot(pc, strict, preferred_element_type=jnp.float32)
        off_ref[...] = off
        ob = off / blk
        jj = lax.broadcasted_iota(jnp.int32, (nblocks, n_exp), 0).astype(jnp.float32)
        be = jnp.sum((jj >= ob).astype(jnp.float32), axis=1, keepdims=True) - 1.0
        be_ref[...] = be.astype(jnp.int32)
        nb_ref[...] = jnp.sum(nblk, axis=1, keepdims=True).astype(jnp.int32)

    @pl.when(ph == 1)
    def _():
        off = off_ref[...]
        pos = jnp.sum(oh * off, axis=1, keepdims=True) + r_ref[pl.ds(j * sblk, sblk), :]
        p_ref[...] = pos.astype(jnp.int32)


def _dispatch_body(pos_hbm, g_hbm, x_hbm, xs_hbm, gs_hbm,
                   pos_v, tok_v, gch_v, gidx_v, gsl_v,
                   rows_a, rows_b, sp_tok, sp_g, sem, *, n_tok, spad, d_model):
    cid = lax.axis_index("c")
    sid = lax.axis_index("s")
    wid = sid * NC + cid

    slots_per_tile = 2 * n_tok // NS
    rows_per_tile = spad // NW
    gchunk = rows_a.shape[0]

    base = sid * slots_per_tile
    pltpu.sync_copy(pos_hbm.at[pl.ds(base, slots_per_tile)], pos_v)
    pltpu.sync_copy(g_hbm.at[pl.ds(base, slots_per_tile)], gch_v)
    for i in range(slots_per_tile // 16):
        s = base + i * 16 + lax.iota(jnp.int32, 16)
        tok_v[pl.ds(i * 16, 16)] = jnp.where(s < n_tok, s, s - n_tok)

    pltpu.sync_copy(tok_v, sp_tok.at[pos_v])
    pltpu.sync_copy(gch_v, sp_g.at[pos_v])
    plsc.subcore_barrier()

    rbase = wid * rows_per_tile
    pltpu.sync_copy(sp_tok.at[pl.ds(rbase, rows_per_tile)], gidx_v)
    pltpu.sync_copy(sp_g.at[pl.ds(rbase, rows_per_tile)], gsl_v)
    pltpu.sync_copy(gsl_v, gs_hbm.at[pl.ds(rbase, rows_per_tile)])
    for i in range(rows_per_tile // 16):
        sl = pl.ds(i * 16, 16)
        gidx_v[sl] = jnp.minimum(jnp.maximum(gidx_v[sl], 0), n_tok - 1)
    nch = rows_per_tile // gchunk
    bufs = (rows_a, rows_b)
    d = pltpu.async_copy(x_hbm.at[gidx_v.at[pl.ds(0, gchunk)]], rows_a, sem)
    for c in range(nch):
        d.wait()
        if c + 1 < nch:
            d = pltpu.async_copy(
                x_hbm.at[gidx_v.at[pl.ds((c + 1) * gchunk, gchunk)]],
                bufs[(c + 1) % 2], sem)
        pltpu.sync_copy(bufs[c % 2], xs_hbm.at[pl.ds(rbase + c * gchunk, gchunk)])


def _ffn_body(be_ref, nb_ref, xs_ref, gs_ref, w1_ref, b1_ref, w2_ref, b2_ref,
              yo_ref, w1b_ref, w2b_ref):
    i = pl.program_id(0)
    active = i < nb_ref[0]
    ecur = be_ref[i]
    eprev = be_ref[jnp.maximum(i - 1, 0)]
    need_cast = jnp.logical_or(i == 0, ecur != eprev)

    @pl.when(active & need_cast)
    def _():
        w1b_ref[...] = w1_ref[0].astype(jnp.bfloat16)
        w2b_ref[...] = w2_ref[0].astype(jnp.bfloat16)

    @pl.when(active)
    def _():
        xb = xs_ref[...].astype(jnp.bfloat16)
        h = jnp.dot(xb, w1b_ref[...], preferred_element_type=jnp.float32) + b1_ref[0, 0]
        hb = jnp.maximum(h, 0.0).astype(jnp.bfloat16)
        eo = jnp.dot(hb, w2b_ref[...], preferred_element_type=jnp.float32) + b2_ref[0, 0]
        yo_ref[...] = eo * gs_ref[...]


def _combine_body(yo_hbm, p0_hbm, p1_hbm, out_hbm, p0_v, p1_v, r0_v, r1_v,
                  sem, *, n_tok):
    cid = lax.axis_index("c")
    sid = lax.axis_index("s")
    wid = sid * NC + cid
    tpt = n_tok // NW
    base = wid * tpt
    half = r0_v.shape[0]
    d_model = r0_v.shape[1]
    pltpu.sync_copy(p0_hbm.at[pl.ds(base, tpt)], p0_v)
    pltpu.sync_copy(p1_hbm.at[pl.ds(base, tpt)], p1_v)
    for c in range(tpt // half):
        d0 = pltpu.async_copy(yo_hbm.at[p0_v.at[pl.ds(c * half, half)]], r0_v, sem)
        d1 = pltpu.async_copy(yo_hbm.at[p1_v.at[pl.ds(c * half, half)]], r1_v, sem)
        d0.wait()
        d1.wait()

        def row(r, _):
            def chunk(k, _):
                sl = pl.ds(k * 16, 16)
                r0_v[r, sl] = r0_v[r, sl] + r1_v[r, sl]
                return 0
            return lax.fori_loop(0, d_model // 16, chunk, 0)

        lax.fori_loop(0, half, row, 0)
        pltpu.sync_copy(r0_v, out_hbm.at[pl.ds(base + c * half, half)])


def kernel(x, Wg, bg, Wn, bn, W1, b1, W2, b2):
    N, D = x.shape
    E = Wg.shape[1]
    H = W1.shape[2]
    S = 2 * N
    NB = S // BLK + E
    SPAD = NB * BLK

    noise = jax.random.normal(jax.random.key(42), (N, E), jnp.float32)

    tb = 256
    e0, e1, g0, g1 = pl.pallas_call(
        _router_body,
        grid=(N // tb,),
        in_specs=[
            pl.BlockSpec((tb, D), lambda t: (t, 0)),
            pl.BlockSpec((D, E), lambda t: (0, 0)),
            pl.BlockSpec((1, E), lambda t: (0, 0)),
            pl.BlockSpec((D, E), lambda t: (0, 0)),
            pl.BlockSpec((1, E), lambda t: (0, 0)),
            pl.BlockSpec((tb, E), lambda t: (t, 0)),
        ],
        out_specs=[
            pl.BlockSpec((tb, 1), lambda t: (t, 0)),
            pl.BlockSpec((tb, 1), lambda t: (t, 0)),
            pl.BlockSpec((tb, 1), lambda t: (t, 0)),
            pl.BlockSpec((tb, 1), lambda t: (t, 0)),
        ],
        out_shape=[
            jax.ShapeDtypeStruct((N, 1), jnp.int32),
            jax.ShapeDtypeStruct((N, 1), jnp.int32),
            jax.ShapeDtypeStruct((N, 1), jnp.float32),
            jax.ShapeDtypeStruct((N, 1), jnp.float32),
        ],
    )(x, Wg, bg.reshape(1, E), Wn, bn.reshape(1, E), noise)

    es = jnp.concatenate([e0, e1], axis=0)
    gall = jnp.concatenate([g0, g1], axis=0)

    sblk = 512
    pos, be, nbact = pl.pallas_call(
        functools.partial(_ranking_body, n_exp=E, blk=BLK, nblocks=NB, sblk=sblk),
        grid=(2, S // sblk),
        in_specs=[pl.BlockSpec((sblk, 1), lambda ph, j: (j, 0))],
        out_specs=[
            pl.BlockSpec((sblk, 1), lambda ph, j: (j, 0)),
            pl.BlockSpec((NB, 1), lambda ph, j: (0, 0)),
            pl.BlockSpec((1, 1), lambda ph, j: (0, 0)),
        ],
        out_shape=[
            jax.ShapeDtypeStruct((S, 1), jnp.int32),
            jax.ShapeDtypeStruct((NB, 1), jnp.int32),
            jax.ShapeDtypeStruct((1, 1), jnp.int32),
        ],
        scratch_shapes=[
            pltpu.VMEM((S, 1), jnp.float32),
            pltpu.VMEM((1, E), jnp.float32),
            pltpu.VMEM((1, E), jnp.float32),
        ],
    )(es)

    pos_flat = pos.reshape(S)
    g_flat = gall.reshape(S)

    mesh = plsc.VectorSubcoreMesh(core_axis_name="c", subcore_axis_name="s",
                                  num_cores=NC, num_subcores=NS)
    xs, gsort = pl.kernel(
        functools.partial(_dispatch_body, n_tok=N, spad=SPAD, d_model=D),
        out_type=[
            jax.ShapeDtypeStruct((SPAD, D), jnp.float32),
            jax.ShapeDtypeStruct((SPAD,), jnp.float32),
        ],
        mesh=mesh,
        scratch_types=[
            pltpu.VMEM((S // NS,), jnp.int32),
            pltpu.VMEM((S // NS,), jnp.int32),
            pltpu.VMEM((S // NS,), jnp.float32),
            pltpu.VMEM((SPAD // NW,), jnp.int32),
            pltpu.VMEM((SPAD // NW,), jnp.float32),
            pltpu.VMEM((SPAD // NW // 4, D), jnp.float32),
            pltpu.VMEM((SPAD // NW // 4, D), jnp.float32),
            pltpu.VMEM_SHARED((SPAD,), jnp.int32),
            pltpu.VMEM_SHARED((SPAD,), jnp.float32),
            pltpu.SemaphoreType.DMA,
        ],
    )(pos_flat, g_flat, x)

    grid_spec = pltpu.PrefetchScalarGridSpec(
        num_scalar_prefetch=2,
        grid=(NB,),
        in_specs=[
            pl.BlockSpec((BLK, D), lambda i, be_r, nb_r: (i, 0)),
            pl.BlockSpec((BLK, 1), lambda i, be_r, nb_r: (i, 0)),
            pl.BlockSpec((1, D, H), lambda i, be_r, nb_r: (be_r[i], 0, 0)),
            pl.BlockSpec((1, 1, H), lambda i, be_r, nb_r: (be_r[i], 0, 0)),
            pl.BlockSpec((1, H, D), lambda i, be_r, nb_r: (be_r[i], 0, 0)),
            pl.BlockSpec((1, 1, D), lambda i, be_r, nb_r: (be_r[i], 0, 0)),
        ],
        out_specs=pl.BlockSpec((BLK, D), lambda i, be_r, nb_r: (i, 0)),
        scratch_shapes=[
            pltpu.VMEM((D, H), jnp.bfloat16),
            pltpu.VMEM((H, D), jnp.bfloat16),
        ],
    )
    yo = pl.pallas_call(
        _ffn_body,
        grid_spec=grid_spec,
        out_shape=jax.ShapeDtypeStruct((SPAD, D), jnp.float32),
    )(be.reshape(NB), nbact.reshape(1), xs, gsort.reshape(SPAD, 1),
      W1, b1.reshape(E, 1, H), W2, b2.reshape(E, 1, D))

    out = pl.kernel(
        functools.partial(_combine_body, n_tok=N),
        out_type=jax.ShapeDtypeStruct((N, D), jnp.float32),
        mesh=mesh,
        scratch_types=[
            pltpu.VMEM((N // NW,), jnp.int32),
            pltpu.VMEM((N // NW,), jnp.int32),
            pltpu.VMEM((N // NW // 2, D), jnp.float32),
            pltpu.VMEM((N // NW // 2, D), jnp.float32),
            pltpu.SemaphoreType.DMA,
        ],
    )(yo, pos_flat[:N], pos_flat[N:])
    return out


# speedup vs baseline: 1.8603x; 1.0622x over previous
import functools

import jax
import jax.numpy as jnp
from jax import lax
from jax.experimental import pallas as pl
from jax.experimental.pallas import tpu as pltpu
from jax.experimental.pallas import tpu_sc as plsc

NC = 2
NS = 16
NW = NC * NS
BLK = 256


def _router_body(x_ref, wg_ref, bg_ref, wn_ref, bn_ref, noise_ref,
                 e0_ref, e1_ref, g0_ref, g1_ref, xpk_ref):
    x = x_ref[...]
    tb, dm = x.shape
    y = jax.lax.bitcast_convert_type(
        x.astype(jnp.bfloat16).astype(jnp.float32), jnp.int32)
    hi = jax.lax.bitwise_and(y[:, :dm // 2], jnp.int32(-65536))
    lo = jax.lax.shift_right_logical(y[:, dm // 2:], 16)
    xpk_ref[...] = jax.lax.bitwise_or(hi, lo)
    logits = jnp.dot(x, wg_ref[...], preferred_element_type=jnp.float32) + bg_ref[...]
    nlog = jnp.dot(x, wn_ref[...], preferred_element_type=jnp.float32) + bn_ref[...]
    sp = jnp.maximum(nlog, 0.0) + jnp.log1p(jnp.exp(-jnp.abs(nlog)))
    noisy = logits + noise_ref[...] * sp

    E = noisy.shape[-1]
    col = lax.broadcasted_iota(jnp.int32, noisy.shape, 1)
    m1 = jnp.max(noisy, axis=1, keepdims=True)
    i1 = jnp.min(jnp.where(noisy == m1, col, E), axis=1, keepdims=True)
    masked = jnp.where(col == i1, -jnp.inf, noisy)
    m2 = jnp.max(masked, axis=1, keepdims=True)
    i2 = jnp.min(jnp.where(masked == m2, col, E), axis=1, keepdims=True)
    w2 = jnp.exp(m2 - m1)
    denom = 1.0 + w2
    e0_ref[...] = i1
    e1_ref[...] = i2
    g0_ref[...] = 1.0 / denom
    g1_ref[...] = w2 / denom


def _ranking_body(es_ref, p_ref, be_ref, nb_ref, r_ref, carry_ref, off_ref,
                  *, n_exp, blk, nblocks, sblk):
    ph = pl.program_id(0)
    j = pl.program_id(1)

    es = es_ref[...]
    lane = lax.broadcasted_iota(jnp.int32, (sblk, n_exp), 1)
    oh = (es == lane).astype(jnp.float32)

    @pl.when(ph == 0)
    def _():
        r = lax.broadcasted_iota(jnp.int32, (sblk, sblk), 0)
        c = lax.broadcasted_iota(jnp.int32, (sblk, sblk), 1)
        tril = (r >= c).astype(jnp.float32)
        csum = jnp.dot(tril, oh, preferred_element_type=jnp.float32)
        carry = jnp.where(j == 0, 0.0, carry_ref[...])
        rank = jnp.sum(oh * (csum + carry), axis=1, keepdims=True) - 1.0
        r_ref[pl.ds(j * sblk, sblk), :] = rank
        carry_ref[...] = carry + csum[sblk - 1:sblk, :]

    @pl.when((ph == 1) & (j == 0))
    def _():
        counts = carry_ref[...]
        nblk = jnp.floor((counts + (blk - 1)) / blk)
        pc = nblk * blk
        a = lax.broadcasted_iota(jnp.int32, (n_exp, n_exp), 0)
        b = lax.broadcasted_iota(jnp.int32, (n_exp, n_exp), 1)
        strict = (a < b).astype(jnp.float32)
        off = jnp.dot(pc, strict, preferred_element_type=jnp.float32)
        off_ref[...] = off
        ob = off / blk
        jj = lax.broadcasted_iota(jnp.int32, (nblocks, n_exp), 0).astype(jnp.float32)
        be = jnp.sum((jj >= ob).astype(jnp.float32), axis=1, keepdims=True) - 1.0
        be_ref[...] = be.astype(jnp.int32)
        nb_ref[...] = jnp.sum(nblk, axis=1, keepdims=True).astype(jnp.int32)

    @pl.when(ph == 1)
    def _():
        off = off_ref[...]
        pos = jnp.sum(oh * off, axis=1, keepdims=True) + r_ref[pl.ds(j * sblk, sblk), :]
        p_ref[...] = pos.astype(jnp.int32)


def _dispatch_body(pos_hbm, g_hbm, x_hbm, xs_hbm, gs_hbm,
                   pos_v, tok_v, gch_v, gidx_v, gsl_v,
                   rows_a, rows_b, sp_tok, sp_g, sem, *, n_tok, spad, d_model):
    cid = lax.axis_index("c")
    sid = lax.axis_index("s")
    wid = sid * NC + cid

    slots_per_tile = 2 * n_tok // NS
    rows_per_tile = spad // NW
    gchunk = rows_a.shape[0]

    base = sid * slots_per_tile
    pltpu.sync_copy(pos_hbm.at[pl.ds(base, slots_per_tile)], pos_v)
    pltpu.sync_copy(g_hbm.at[pl.ds(base, slots_per_tile)], gch_v)
    for i in range(slots_per_tile // 16):
        s = base + i * 16 + lax.iota(jnp.int32, 16)
        tok_v[pl.ds(i * 16, 16)] = jnp.where(s < n_tok, s, s - n_tok)

    pltpu.sync_copy(tok_v, sp_tok.at[pos_v])
    pltpu.sync_copy(gch_v, sp_g.at[pos_v])
    plsc.subcore_barrier()

    rbase = wid * rows_per_tile
    pltpu.sync_copy(sp_tok.at[pl.ds(rbase, rows_per_tile)], gidx_v)
    pltpu.sync_copy(sp_g.at[pl.ds(rbase, rows_per_tile)], gsl_v)
    pltpu.sync_copy(gsl_v, gs_hbm.at[pl.ds(rbase, rows_per_tile)])
    for i in range(rows_per_tile // 16):
        sl = pl.ds(i * 16, 16)
        gidx_v[sl] = jnp.minimum(jnp.maximum(gidx_v[sl], 0), n_tok - 1)
    nch = rows_per_tile // gchunk
    bufs = (rows_a, rows_b)
    d = pltpu.async_copy(x_hbm.at[gidx_v.at[pl.ds(0, gchunk)]], rows_a, sem)
    for c in range(nch):
        d.wait()
        if c + 1 < nch:
            d = pltpu.async_copy(
                x_hbm.at[gidx_v.at[pl.ds((c + 1) * gchunk, gchunk)]],
                bufs[(c + 1) % 2], sem)
        pltpu.sync_copy(bufs[c % 2], xs_hbm.at[pl.ds(rbase + c * gchunk, gchunk)])


def _ffn_body(be_ref, nb_ref, xs_ref, gs_ref, w1_ref, b1_ref, w2_ref, b2_ref,
              yo_ref, w1b_ref, w2b_ref):
    i = pl.program_id(0)
    active = i < nb_ref[0]
    ecur = be_ref[i]
    eprev = be_ref[jnp.maximum(i - 1, 0)]
    need_cast = jnp.logical_or(i == 0, ecur != eprev)

    @pl.when(active & need_cast)
    def _():
        w1b_ref[...] = w1_ref[0].astype(jnp.bfloat16)
        w2b_ref[...] = w2_ref[0].astype(jnp.bfloat16)

    @pl.when(active)
    def _():
        xw = xs_ref[...]
        a = jax.lax.bitcast_convert_type(
            jax.lax.bitwise_and(xw, jnp.int32(-65536)), jnp.float32)
        b = jax.lax.bitcast_convert_type(
            jax.lax.shift_left(xw, 16), jnp.float32)
        xb = jnp.concatenate([a, b], axis=1).astype(jnp.bfloat16)
        h = jnp.dot(xb, w1b_ref[...], preferred_element_type=jnp.float32) + b1_ref[0, 0]
        hb = jnp.maximum(h, 0.0).astype(jnp.bfloat16)
        eo = jnp.dot(hb, w2b_ref[...], preferred_element_type=jnp.float32) + b2_ref[0, 0]
        yo_ref[...] = eo * gs_ref[...]


def _combine_body(yo_hbm, p0_hbm, p1_hbm, out_hbm, p0_v, p1_v, r0_v, r1_v,
                  sem, *, n_tok):
    cid = lax.axis_index("c")
    sid = lax.axis_index("s")
    wid = sid * NC + cid
    tpt = n_tok // NW
    base = wid * tpt
    half = r0_v.shape[0]
    d_model = r0_v.shape[1]
    pltpu.sync_copy(p0_hbm.at[pl.ds(base, tpt)], p0_v)
    pltpu.sync_copy(p1_hbm.at[pl.ds(base, tpt)], p1_v)
    for c in range(tpt // half):
        d0 = pltpu.async_copy(yo_hbm.at[p0_v.at[pl.ds(c * half, half)]], r0_v, sem)
        d1 = pltpu.async_copy(yo_hbm.at[p1_v.at[pl.ds(c * half, half)]], r1_v, sem)
        d0.wait()
        d1.wait()

        def row(r, _):
            def chunk(k, _):
                sl = pl.ds(k * 16, 16)
                r0_v[r, sl] = r0_v[r, sl] + r1_v[r, sl]
                return 0
            return lax.fori_loop(0, d_model // 16, chunk, 0)

        lax.fori_loop(0, half, row, 0)
        pltpu.sync_copy(r0_v, out_hbm.at[pl.ds(base + c * half, half)])


def kernel(x, Wg, bg, Wn, bn, W1, b1, W2, b2):
    N, D = x.shape
    E = Wg.shape[1]
    H = W1.shape[2]
    S = 2 * N
    NB = S // BLK + E
    SPAD = NB * BLK

    noise = jax.random.normal(jax.random.key(42), (N, E), jnp.float32)

    tb = 256
    e0, e1, g0, g1, xpk = pl.pallas_call(
        _router_body,
        grid=(N // tb,),
        in_specs=[
            pl.BlockSpec((tb, D), lambda t: (t, 0)),
            pl.BlockSpec((D, E), lambda t: (0, 0)),
            pl.BlockSpec((1, E), lambda t: (0, 0)),
            pl.BlockSpec((D, E), lambda t: (0, 0)),
            pl.BlockSpec((1, E), lambda t: (0, 0)),
            pl.BlockSpec((tb, E), lambda t: (t, 0)),
        ],
        out_specs=[
            pl.BlockSpec((tb, 1), lambda t: (t, 0)),
            pl.BlockSpec((tb, 1), lambda t: (t, 0)),
            pl.BlockSpec((tb, 1), lambda t: (t, 0)),
            pl.BlockSpec((tb, 1), lambda t: (t, 0)),
            pl.BlockSpec((tb, D // 2), lambda t: (t, 0)),
        ],
        out_shape=[
            jax.ShapeDtypeStruct((N, 1), jnp.int32),
            jax.ShapeDtypeStruct((N, 1), jnp.int32),
            jax.ShapeDtypeStruct((N, 1), jnp.float32),
            jax.ShapeDtypeStruct((N, 1), jnp.float32),
            jax.ShapeDtypeStruct((N, D // 2), jnp.int32),
        ],
    )(x, Wg, bg.reshape(1, E), Wn, bn.reshape(1, E), noise)

    es = jnp.concatenate([e0, e1], axis=0)
    gall = jnp.concatenate([g0, g1], axis=0)

    sblk = 512
    pos, be, nbact = pl.pallas_call(
        functools.partial(_ranking_body, n_exp=E, blk=BLK, nblocks=NB, sblk=sblk),
        grid=(2, S // sblk),
        in_specs=[pl.BlockSpec((sblk, 1), lambda ph, j: (j, 0))],
        out_specs=[
            pl.BlockSpec((sblk, 1), lambda ph, j: (j, 0)),
            pl.BlockSpec((NB, 1), lambda ph, j: (0, 0)),
            pl.BlockSpec((1, 1), lambda ph, j: (0, 0)),
        ],
        out_shape=[
            jax.ShapeDtypeStruct((S, 1), jnp.int32),
            jax.ShapeDtypeStruct((NB, 1), jnp.int32),
            jax.ShapeDtypeStruct((1, 1), jnp.int32),
        ],
        scratch_shapes=[
            pltpu.VMEM((S, 1), jnp.float32),
            pltpu.VMEM((1, E), jnp.float32),
            pltpu.VMEM((1, E), jnp.float32),
        ],
    )(es)

    pos_flat = pos.reshape(S)
    g_flat = gall.reshape(S)

    mesh = plsc.VectorSubcoreMesh(core_axis_name="c", subcore_axis_name="s",
                                  num_cores=NC, num_subcores=NS)
    xs, gsort = pl.kernel(
        functools.partial(_dispatch_body, n_tok=N, spad=SPAD, d_model=D),
        out_type=[
            jax.ShapeDtypeStruct((SPAD, D // 2), jnp.int32),
            jax.ShapeDtypeStruct((SPAD,), jnp.float32),
        ],
        mesh=mesh,
        scratch_types=[
            pltpu.VMEM((S // NS,), jnp.int32),
            pltpu.VMEM((S // NS,), jnp.int32),
            pltpu.VMEM((S // NS,), jnp.float32),
            pltpu.VMEM((SPAD // NW,), jnp.int32),
            pltpu.VMEM((SPAD // NW,), jnp.float32),
            pltpu.VMEM((SPAD // NW // 4, D // 2), jnp.int32),
            pltpu.VMEM((SPAD // NW // 4, D // 2), jnp.int32),
            pltpu.VMEM_SHARED((SPAD,), jnp.int32),
            pltpu.VMEM_SHARED((SPAD,), jnp.float32),
            pltpu.SemaphoreType.DMA,
        ],
    )(pos_flat, g_flat, xpk)

    grid_spec = pltpu.PrefetchScalarGridSpec(
        num_scalar_prefetch=2,
        grid=(NB,),
        in_specs=[
            pl.BlockSpec((BLK, D // 2), lambda i, be_r, nb_r: (i, 0)),
            pl.BlockSpec((BLK, 1), lambda i, be_r, nb_r: (i, 0)),
            pl.BlockSpec((1, D, H), lambda i, be_r, nb_r: (be_r[i], 0, 0)),
            pl.BlockSpec((1, 1, H), lambda i, be_r, nb_r: (be_r[i], 0, 0)),
            pl.BlockSpec((1, H, D), lambda i, be_r, nb_r: (be_r[i], 0, 0)),
            pl.BlockSpec((1, 1, D), lambda i, be_r, nb_r: (be_r[i], 0, 0)),
        ],
        out_specs=pl.BlockSpec((BLK, D), lambda i, be_r, nb_r: (i, 0)),
        scratch_shapes=[
            pltpu.VMEM((D, H), jnp.bfloat16),
            pltpu.VMEM((H, D), jnp.bfloat16),
        ],
    )
    yo = pl.pallas_call(
        _ffn_body,
        grid_spec=grid_spec,
        out_shape=jax.ShapeDtypeStruct((SPAD, D), jnp.float32),
    )(be.reshape(NB), nbact.reshape(1), xs, gsort.reshape(SPAD, 1),
      W1, b1.reshape(E, 1, H), W2, b2.reshape(E, 1, D))

    out = pl.kernel(
        functools.partial(_combine_body, n_tok=N),
        out_type=jax.ShapeDtypeStruct((N, D), jnp.float32),
        mesh=mesh,
        scratch_types=[
            pltpu.VMEM((N // NW,), jnp.int32),
            pltpu.VMEM((N // NW,), jnp.int32),
            pltpu.VMEM((N // NW // 2, D), jnp.float32),
            pltpu.VMEM((N // NW // 2, D), jnp.float32),
            pltpu.SemaphoreType.DMA,
        ],
    )(yo, pos_flat[:N], pos_flat[N:])
    return out
